# baseline (speedup 1.0000x reference)
import functools

import jax
import jax.numpy as jnp
from jax import lax
from jax.experimental import pallas as pl
from jax.experimental.pallas import tpu as pltpu
from jax.experimental.pallas import tpu_sc as plsc

_EPS = 1e-5
_INV_SQRT2 = 0.7071067811865476


def _layer_norm(v, g, b):
    mu = jnp.mean(v, axis=-1, keepdims=True)
    var = jnp.mean((v - mu) ** 2, axis=-1, keepdims=True)
    return (v - mu) * lax.rsqrt(var + _EPS) * g + b


def _gelu(u):
    return u * 0.5 * (1.0 + lax.erf(u * _INV_SQRT2))


def _precompute_body(x_ref, wn_ref, wd_ref, a_ref, c_ref):
    xb = x_ref[...]
    a_ref[...] = jnp.dot(
        xb, wn_ref[...], preferred_element_type=jnp.float32
    ).astype(jnp.bfloat16)
    c_ref[...] = jnp.dot(xb, wd_ref[...], preferred_element_type=jnp.float32)


def _precompute(x2, wn, wd, block_rows):
    n, d = x2.shape
    f = wn.shape[1]
    grid = (n // block_rows,)
    out_spec = pl.BlockSpec((block_rows, f), lambda i: (i, 0))
    return pl.pallas_call(
        _precompute_body,
        grid=grid,
        in_specs=[
            pl.BlockSpec((block_rows, d), lambda i: (i, 0)),
            pl.BlockSpec((d, f), lambda i: (0, 0)),
            pl.BlockSpec((d, f), lambda i: (0, 0)),
        ],
        out_specs=[out_spec, out_spec],
        out_shape=[
            jax.ShapeDtypeStruct((n, f), jnp.bfloat16),
            jax.ShapeDtypeStruct((n, f), jnp.float32),
        ],
    )(x2, wn, wd)


def _sc_gather(table, idx_flat, window):
    n_rows, f = table.shape
    e = idx_flat.shape[0]
    idx2 = idx_flat.reshape(1, e)
    mesh = plsc.VectorSubcoreMesh(core_axis_name="core", subcore_axis_name="subcore")

    @functools.partial(
        pl.kernel,
        out_type=jax.ShapeDtypeStruct((e, f), table.dtype),
        mesh=mesh,
        scratch_types=[pltpu.VMEM_SHARED((n_rows, f), table.dtype)],
        compiler_params=pltpu.CompilerParams(use_tc_tiling_on_sc=False),
    )
    def gather_kernel(tab_hbm, i_hbm, o_hbm, tab_sh):
        @pl.when(lax.axis_index("subcore") == 0)
        def _():
            pltpu.sync_copy(tab_hbm, tab_sh)

        plsc.subcore_barrier()

        def body(i_vmem, o_vmem):
            pltpu.sync_copy(tab_sh.at[i_vmem.at[0]], o_vmem)

        pltpu.emit_pipeline(
            body,
            grid=(e // window,),
            in_specs=[pl.BlockSpec((1, window), index_map=lambda i: (0, i))],
            out_specs=[pl.BlockSpec((window, f), index_map=lambda i: (i, 0))],
            core_axis_name=("core", "subcore"),
            dimension_semantics=(pltpu.PARALLEL,),
        )(i_hbm, o_hbm)

    return gather_kernel(table, idx2)


def _mlp_body(g_ref, c_ref, w2t_ref, g1_ref, b1_ref, g2_ref, b2_ref, o_ref):
    bn, k, f = g_ref.shape
    v = g_ref[...].astype(jnp.float32) + c_ref[...][:, None, :]
    y = _gelu(_layer_norm(v, g1_ref[...], b1_ref[...]))
    h = jnp.dot(
        y.reshape(bn * k, f), w2t_ref[...], preferred_element_type=jnp.float32
    )
    z = _gelu(_layer_norm(h, g2_ref[...], b2_ref[...]))
    o_ref[...] = jnp.max(z.reshape(bn, k, f), axis=1)


def _mlp_max(g3, c, w2t, g1, b1, g2, b2, block_nodes):
    n, k, f = g3.shape
    grid = (n // block_nodes,)
    return pl.pallas_call(
        _mlp_body,
        grid=grid,
        in_specs=[
            pl.BlockSpec((block_nodes, k, f), lambda i: (i, 0, 0)),
            pl.BlockSpec((block_nodes, f), lambda i: (i, 0)),
            pl.BlockSpec((f, f), lambda i: (0, 0)),
            pl.BlockSpec((1, 1, f), lambda i: (0, 0, 0)),
            pl.BlockSpec((1, 1, f), lambda i: (0, 0, 0)),
            pl.BlockSpec((1, f), lambda i: (0, 0)),
            pl.BlockSpec((1, f), lambda i: (0, 0)),
        ],
        out_specs=pl.BlockSpec((block_nodes, f), lambda i: (i, 0)),
        out_shape=jax.ShapeDtypeStruct((n, f), jnp.float32),
    )(g3, c, w2t, g1.reshape(1, 1, f), b1.reshape(1, 1, f),
      g2.reshape(1, f), b2.reshape(1, f))


def kernel(x, neighbor_ind, W1, g1, b1, W2, g2, b2):
    b, n, d = x.shape
    k = neighbor_ind.shape[-1]
    f = W1.shape[0]
    x2 = x.reshape(n, d)
    idx_flat = neighbor_ind.reshape(n * k)

    wn = W1[:, :d].T
    wd = (W1[:, d:] - W1[:, :d]).T
    w2t = W2.T

    a_tab, c_tab = _precompute(x2, wn, wd, block_rows=2000)
    g_flat = _sc_gather(a_tab, idx_flat, window=256)
    g3 = g_flat.reshape(n, k, f)
    out = _mlp_max(g3, c_tab, w2t, g1, b1, g2, b2, block_nodes=1000)
    return out.reshape(b, n, f)


# speedup vs baseline: 1.0961x; 1.0961x over previous
import functools

import jax
import jax.numpy as jnp
from jax import lax
from jax.experimental import pallas as pl
from jax.experimental.pallas import tpu as pltpu
from jax.experimental.pallas import tpu_sc as plsc

_EPS = 1e-5
_INV_SQRT2 = 0.7071067811865476


def _layer_norm(v, g, b):
    mu = jnp.mean(v, axis=-1, keepdims=True)
    var = jnp.mean((v - mu) ** 2, axis=-1, keepdims=True)
    return (v - mu) * lax.rsqrt(var + _EPS) * g + b


def _gelu(u):
    return u * 0.5 * (1.0 + lax.erf(u * _INV_SQRT2))


def _precompute_body(x_ref, wn_ref, wd_ref, a_ref, c_ref):
    xb = x_ref[...]
    a_ref[...] = jnp.dot(xb, wn_ref[...], preferred_element_type=jnp.float32)
    c_ref[...] = jnp.dot(xb, wd_ref[...], preferred_element_type=jnp.float32)


def _precompute(x2, wn, wd, block_rows):
    n, d = x2.shape
    f = wn.shape[1]
    grid = (n // block_rows,)
    out_spec = pl.BlockSpec((block_rows, f), lambda i: (i, 0))
    return pl.pallas_call(
        _precompute_body,
        grid=grid,
        in_specs=[
            pl.BlockSpec((block_rows, d), lambda i: (i, 0)),
            pl.BlockSpec((d, f), lambda i: (0, 0)),
            pl.BlockSpec((d, f), lambda i: (0, 0)),
        ],
        out_specs=[out_spec, out_spec],
        out_shape=[
            jax.ShapeDtypeStruct((n, f), jnp.float32),
            jax.ShapeDtypeStruct((n, f), jnp.float32),
        ],
    )(x2, wn, wd)


def _sc_gather(table, idx_flat, window):
    n_rows, f = table.shape
    e = idx_flat.shape[0]
    idx2 = idx_flat.reshape(1, e)
    mesh = plsc.VectorSubcoreMesh(core_axis_name="core", subcore_axis_name="subcore")

    @functools.partial(
        pl.kernel,
        out_type=jax.ShapeDtypeStruct((e, f), table.dtype),
        mesh=mesh,
        compiler_params=pltpu.CompilerParams(use_tc_tiling_on_sc=False),
    )
    def gather_kernel(tab_hbm, i_hbm, o_hbm):
        def body(i_vmem, o_vmem):
            pltpu.sync_copy(tab_hbm.at[i_vmem.at[0]], o_vmem)

        pltpu.emit_pipeline(
            body,
            grid=(e // window,),
            in_specs=[pl.BlockSpec((1, window), index_map=lambda i: (0, i))],
            out_specs=[pl.BlockSpec((window, f), index_map=lambda i: (i, 0))],
            core_axis_name=("core", "subcore"),
            dimension_semantics=(pltpu.PARALLEL,),
        )(i_hbm, o_hbm)

    return gather_kernel(table, idx2)


def _mlp_body(g_ref, c_ref, w2t_ref, g1_ref, b1_ref, g2_ref, b2_ref, o_ref):
    bn, k, f = g_ref.shape
    v = g_ref[...] + c_ref[...][:, None, :]
    y = _gelu(_layer_norm(v, g1_ref[...], b1_ref[...]))
    h = jnp.dot(
        y.reshape(bn * k, f), w2t_ref[...], preferred_element_type=jnp.float32
    )
    z = _gelu(_layer_norm(h, g2_ref[...], b2_ref[...]))
    o_ref[...] = jnp.max(z.reshape(bn, k, f), axis=1)


def _mlp_max(g3, c, w2t, g1, b1, g2, b2, block_nodes):
    n, k, f = g3.shape
    grid = (n // block_nodes,)
    return pl.pallas_call(
        _mlp_body,
        grid=grid,
        in_specs=[
            pl.BlockSpec((block_nodes, k, f), lambda i: (i, 0, 0)),
            pl.BlockSpec((block_nodes, f), lambda i: (i, 0)),
            pl.BlockSpec((f, f), lambda i: (0, 0)),
            pl.BlockSpec((1, 1, f), lambda i: (0, 0, 0)),
            pl.BlockSpec((1, 1, f), lambda i: (0, 0, 0)),
            pl.BlockSpec((1, f), lambda i: (0, 0)),
            pl.BlockSpec((1, f), lambda i: (0, 0)),
        ],
        out_specs=pl.BlockSpec((block_nodes, f), lambda i: (i, 0)),
        out_shape=jax.ShapeDtypeStruct((n, f), jnp.float32),
    )(g3, c, w2t, g1.reshape(1, 1, f), b1.reshape(1, 1, f),
      g2.reshape(1, f), b2.reshape(1, f))


def kernel(x, neighbor_ind, W1, g1, b1, W2, g2, b2):
    b, n, d = x.shape
    k = neighbor_ind.shape[-1]
    f = W1.shape[0]
    x2 = x.reshape(n, d)
    idx_flat = neighbor_ind.reshape(n * k)

    wn = W1[:, :d].T
    wd = (W1[:, d:] - W1[:, :d]).T
    w2t = W2.T

    a_tab, c_tab = _precompute(x2, wn, wd, block_rows=2000)

    n_chunks = 5
    nc = n // n_chunks
    outs = []
    for c in range(n_chunks):
        idx_c = lax.dynamic_slice_in_dim(idx_flat, c * nc * k, nc * k)
        c_c = lax.dynamic_slice_in_dim(c_tab, c * nc, nc)
        g_c = _sc_gather(a_tab, idx_c, window=640).reshape(nc, k, f)
        outs.append(_mlp_max(g_c, c_c, w2t, g1, b1, g2, b2, block_nodes=1000))
    out = jnp.concatenate(outs, axis=0)
    return out.reshape(b, n, f)


# speedup vs baseline: 1.6889x; 1.5409x over previous
import functools

import jax
import jax.numpy as jnp
from jax import lax
from jax.experimental import pallas as pl
from jax.experimental.pallas import tpu as pltpu
from jax.experimental.pallas import tpu_sc as plsc

_EPS = 1e-5
_INV_SQRT2 = 0.7071067811865476


def _gelu(u):
    return u * 0.5 * (1.0 + lax.erf(u * _INV_SQRT2))


def _precompute_body(x_ref, wnn_ref, wdd_ref, a_ref, c_ref):
    xb = x_ref[...]
    a_ref[...] = jnp.dot(xb, wnn_ref[...], preferred_element_type=jnp.float32)
    c_ref[...] = jnp.dot(xb, wdd_ref[...], preferred_element_type=jnp.float32)


def _precompute(x2, wnn, wdd, block_rows):
    n, d = x2.shape
    w = wnn.shape[1]
    grid = (n // block_rows,)
    out_spec = pl.BlockSpec((block_rows, w), lambda i: (i, 0))
    return pl.pallas_call(
        _precompute_body,
        grid=grid,
        in_specs=[
            pl.BlockSpec((block_rows, d), lambda i: (i, 0)),
            pl.BlockSpec((d, w), lambda i: (0, 0)),
            pl.BlockSpec((d, w), lambda i: (0, 0)),
        ],
        out_specs=[out_spec, out_spec],
        out_shape=[
            jax.ShapeDtypeStruct((n, w), jnp.float32),
            jax.ShapeDtypeStruct((n, w), jnp.float32),
        ],
    )(x2, wnn, wdd)


def _sc_gather(table, idx_flat, window):
    n_rows, w = table.shape
    e = idx_flat.shape[0]
    idx2 = idx_flat.reshape(1, e)
    mesh = plsc.VectorSubcoreMesh(core_axis_name="core", subcore_axis_name="subcore")

    @functools.partial(
        pl.kernel,
        out_type=jax.ShapeDtypeStruct((e, w), table.dtype),
        mesh=mesh,
    )
    def gather_kernel(tab_hbm, i_hbm, o_hbm):
        def body(i_vmem, o_vmem):
            pltpu.sync_copy(tab_hbm.at[i_vmem.at[0]], o_vmem)

        pltpu.emit_pipeline(
            body,
            grid=(e // window,),
            in_specs=[pl.BlockSpec((1, window), index_map=lambda i: (0, i))],
            out_specs=[pl.BlockSpec((window, w), index_map=lambda i: (i, 0))],
            core_axis_name=("core", "subcore"),
            dimension_semantics=(pltpu.PARALLEL,),
        )(i_hbm, o_hbm)

    return gather_kernel(table, idx2)


def _mlp_body(g_ref, c_ref, s_ref, w2d_ref, g1_ref, b1_ref, g2_ref, b2_ref,
              o_ref):
    rows, w = g_ref.shape
    bn = c_ref.shape[0]
    sub = rows // bn
    s_mat = s_ref[...]

    def ln(u, g, b):
        mu = jnp.dot(u, s_mat, preferred_element_type=jnp.float32)
        q = jnp.dot(u * u, s_mat, preferred_element_type=jnp.float32)
        var = q - mu * mu
        return (u - mu) * lax.rsqrt(var + _EPS) * g + b

    g3 = g_ref[...].reshape(bn, sub, w)
    v = (g3 + c_ref[...][:, None, :]).reshape(rows, w)
    y = _gelu(ln(v, g1_ref[...], b1_ref[...]))
    h = jnp.dot(y, w2d_ref[...], preferred_element_type=jnp.float32)
    z = _gelu(ln(h, g2_ref[...], b2_ref[...]))
    m = jnp.max(z.reshape(bn, sub, w), axis=1)
    o_ref[...] = jnp.maximum(m[:, : w // 2], m[:, w // 2:])


def _mlp_max(g2d, c2, s_mat, w2d, g1x, b1x, g2x, b2x, block_nodes):
    rows, w = g2d.shape
    n = c2.shape[0]
    k = rows // n
    f = w // 2
    grid = (n // block_nodes,)
    return pl.pallas_call(
        _mlp_body,
        grid=grid,
        in_specs=[
            pl.BlockSpec((block_nodes * k, w), lambda i: (i, 0)),
            pl.BlockSpec((block_nodes, w), lambda i: (i, 0)),
            pl.BlockSpec((w, w), lambda i: (0, 0)),
            pl.BlockSpec((w, w), lambda i: (0, 0)),
            pl.BlockSpec((1, w), lambda i: (0, 0)),
            pl.BlockSpec((1, w), lambda i: (0, 0)),
            pl.BlockSpec((1, w), lambda i: (0, 0)),
            pl.BlockSpec((1, w), lambda i: (0, 0)),
        ],
        out_specs=pl.BlockSpec((block_nodes, f), lambda i: (i, 0)),
        out_shape=jax.ShapeDtypeStruct((n, f), jnp.float32),
    )(g2d, c2, s_mat, w2d, g1x, b1x, g2x, b2x)


def kernel(x, neighbor_ind, W1, g1, b1, W2, g2, b2):
    b, n, d = x.shape
    k = neighbor_ind.shape[-1]
    f = W1.shape[0]
    x2 = x.reshape(n, d)
    idx_flat = neighbor_ind.reshape(n * k)

    wn = W1[:, :d].T
    wd = (W1[:, d:] - W1[:, :d]).T
    wnn = jnp.concatenate([wn, wn], axis=1)
    wdd = jnp.concatenate([wd, wd], axis=1)
    w2t = W2.T
    zero = jnp.zeros_like(w2t)
    w2d = jnp.concatenate(
        [jnp.concatenate([w2t, zero], axis=1),
         jnp.concatenate([zero, w2t], axis=1)], axis=0)

    half = (jnp.arange(2 * f) >= f).astype(jnp.float32)
    same_half = half[:, None] * half[None, :] + (1 - half[:, None]) * (
        1 - half[None, :])
    s_mat = same_half / f

    g1x = jnp.tile(g1, 2).reshape(1, 2 * f)
    b1x = jnp.tile(b1, 2).reshape(1, 2 * f)
    g2x = jnp.tile(g2, 2).reshape(1, 2 * f)
    b2x = jnp.tile(b2, 2).reshape(1, 2 * f)

    a2_tab, c2_tab = _precompute(x2, wnn, wdd, block_rows=2000)

    n_chunks = 5
    nc = n // n_chunks
    outs = []
    for c in range(n_chunks):
        idx_c = lax.dynamic_slice_in_dim(idx_flat, c * nc * k, nc * k)
        c_c = lax.dynamic_slice_in_dim(c2_tab, c * nc, nc)
        g2d = _sc_gather(a2_tab, idx_c, window=256)
        outs.append(
            _mlp_max(g2d, c_c, s_mat, w2d, g1x, b1x, g2x, b2x,
                     block_nodes=400))
    out = jnp.concatenate(outs, axis=0)
    return out.reshape(b, n, f)
